# baseline (speedup 1.0000x reference)
import jax
import jax.numpy as jnp
from jax.experimental import pallas as pl
from jax.experimental.pallas import tpu as pltpu

B, P, NF, NC = 128, 512, 3, 6
K = 8
GNN_DIM, HEADS = 5, 3
GPB = 4


def _leaky(v):
    return jnp.maximum(v, 0.2 * v)


def _mm(a, b):
    return jnp.dot(a, b, preferred_element_type=jnp.float32)


def _dg(a, b):
    return jax.lax.dot_general(
        a, b, (((1,), (1,)), ((), ())), preferred_element_type=jnp.float32)


def _graph_tail(xg, cond, r, w):
    (cW0, cb0, cW1, cb1, cW2, cb2,
     rW0, rb0, rW1, rb1,
     lW0, lb0, lW1, lb1,
     g1W0, g1b0, g1W1, g1b1,
     gWl, gWr, attT, gb,
     g2W0, g2b0, g2W1, g2b1,
     fW0, fb0, fW1, fb1) = w
    ones_col = jnp.ones((P, 1), jnp.float32)

    def col_b(v):
        return _dg(v, ones_col)

    def addpq(a, b):
        return _dg(jnp.concatenate([a, ones_col], axis=1),
                   jnp.concatenate([ones_col, b], axis=1))

    sq = jnp.sum(xg * xg, axis=1, keepdims=True)
    xgT = xg.T
    sq_row = jnp.sum(xgT * xgT, axis=0, keepdims=True)
    gram2 = _dg(xg, -2.0 * xg)
    d2 = (col_b(sq) + gram2) + sq_row
    iota_f = jax.lax.broadcasted_iota(jnp.int32, (P, P), 1).astype(jnp.float32)
    inf = jnp.float32(jnp.inf)
    d2m = d2
    for _ in range(K):
        idxf = jnp.argmin(d2m, axis=1, keepdims=True).astype(jnp.float32)
        sel = iota_f == col_b(idxf)
        d2m = jnp.where(sel, inf, d2m)
    bool_mask = d2m == inf
    maskf = jnp.where(bool_mask, 1.0, 0.0)

    y = xg + _mm(maskf, xg)
    y = _leaky(_mm(y, g1W0) + g1b0)
    xg = _mm(y, g1W1) + g1b1

    xl = _mm(xg, gWl)
    xr = _mm(xg, gWr)
    att4 = 0.4 * attT
    heads = []
    for hh in range(HEADS):
        lo = hh * GNN_DIM
        xl_h = xl[:, lo:lo + GNN_DIM]
        xr_h = xr[:, lo:lo + GNN_DIM]
        a_col = attT[:, hh:hh + 1]
        sr6 = 0.6 * _mm(xr_h, a_col)
        sl6 = 0.6 * _mm(xl_h, a_col)
        e = addpq(sr6, sl6)
        for dd in range(GNN_DIM):
            c = lo + dd
            z = addpq(xr[:, c:c + 1], xl[:, c:c + 1])
            e = e + att4[dd:dd + 1, hh:hh + 1] * jnp.abs(z)
        e_masked = jnp.where(bool_mask, e, jnp.float32(-1e30))
        emax = jnp.max(e_masked, axis=1, keepdims=True)
        ee = jnp.exp(e_masked - col_b(emax))
        den = jnp.sum(ee, axis=1, keepdims=True)
        rec = 1.0 / (den + 1e-16)
        heads.append(rec * _mm(ee, xl_h))
    xg = jnp.concatenate(heads, axis=1) + gb

    y = xg + _mm(maskf, xg)
    y = _leaky(_mm(y, g2W0) + g2b0)
    xg = _mm(y, g2W1) + g2b1

    gnn = jnp.sum(xg, axis=0, keepdims=True)
    fin = jnp.concatenate([r, cond, gnn], axis=1)
    o = _leaky(_mm(fin, fW0) + fb0)
    o = _mm(o, fW1) + fb1
    return o


def _block_kernel(x_ref, cond_ref, *refs):
    out_ref = refs[-1]
    w = tuple(ref[...] for ref in refs[:-1])
    (cW0, cb0, cW1, cb1, cW2, cb2,
     rW0, rb0, rW1, rb1,
     lW0, lb0, lW1, lb1) = w[:14]

    x_all = x_ref[...]
    f = x_all
    for Wt, b in ((cW0, cb0), (cW1, cb1), (cW2, cb2)):
        f = _leaky(_mm(f, Wt) + b)
    conds, rs, cond_rows, r_rows = [], [], [], []
    for g in range(GPB):
        feat = jnp.max(f[g * P:(g + 1) * P, :], axis=0, keepdims=True)
        r = _leaky(_mm(feat, rW0) + rb0)
        r = _mm(r, rW1) + rb1
        cond = cond_ref[g].reshape(1, NC)
        conds.append(cond)
        rs.append(r)
        cond_rows.append(jnp.broadcast_to(cond, (P, NC)))
        r_rows.append(jnp.broadcast_to(r, (P, r.shape[1])))
    h = jnp.concatenate(
        [x_all,
         jnp.concatenate(cond_rows, axis=0),
         jnp.concatenate(r_rows, axis=0)], axis=1)
    xg_all = _leaky(_mm(h, lW0) + lb0)
    xg_all = _mm(xg_all, lW1) + lb1

    outs = []
    for g in range(GPB):
        xg = xg_all[g * P:(g + 1) * P, :]
        outs.append(_graph_tail(xg, conds[g], rs[g], w).reshape(1, 1, 1))
    out_ref[...] = jnp.concatenate(outs, axis=0)


def _flatten_params(params):
    flat = []

    def lin(layers):
        for W, b in layers:
            flat.append(W.T)
            flat.append(b.reshape(1, -1))

    lin(params["conv"])
    lin(params["rgan"])
    lin(params["lin"])
    lin(params["gin1"])
    gp = params["gat"]
    flat += [gp["Wl"].T, gp["Wr"].T, gp["att"].T, gp["b"].reshape(1, -1)]
    lin(params["gin2"])
    lin(params["final"])
    return flat


def kernel(x, cond, params, batch_ids):
    del batch_ids
    flat = _flatten_params(params)

    weight_specs = [
        pl.BlockSpec(w.shape, lambda b, _r=w.ndim: (0,) * _r)
        for w in flat
    ]
    out = pl.pallas_call(
        _block_kernel,
        grid=(B // GPB,),
        in_specs=[
            pl.BlockSpec((GPB * P, NF), lambda b: (b, 0)),
            pl.BlockSpec((GPB, 1, NC), lambda b: (b, 0, 0)),
            *weight_specs,
        ],
        out_specs=pl.BlockSpec((GPB, 1, 1), lambda b: (b, 0, 0)),
        out_shape=jax.ShapeDtypeStruct((B, 1, 1), jnp.float32),
        compiler_params=pltpu.CompilerParams(
            dimension_semantics=("parallel",),
        ),
    )(x, cond.reshape(B, 1, NC), *flat)
    return out.reshape(B)


# speedup vs baseline: 1.2477x; 1.2477x over previous
import jax
import jax.numpy as jnp
from jax.experimental import pallas as pl
from jax.experimental.pallas import tpu as pltpu

B, P, NF, NC = 128, 512, 3, 6
K = 8
GNN_DIM, HEADS = 5, 3
GPB = 2


def _leaky(v):
    return jnp.maximum(v, 0.2 * v)


def _mm(a, b):
    return jnp.dot(a, b, preferred_element_type=jnp.float32)


def _dg(a, b):
    return jax.lax.dot_general(
        a, b, (((1,), (1,)), ((), ())), preferred_element_type=jnp.float32)


def _graph_tail(xg, cond, r, w):
    (cW0, cb0, cW1, cb1, cW2, cb2,
     rW0, rb0, rW1, rb1,
     lW0, lb0, lW1, lb1,
     g1W0, g1b0, g1W1, g1b1,
     gWl, gWr, attT, gb,
     g2W0, g2b0, g2W1, g2b1,
     fW0, fb0, fW1, fb1) = w
    ones_col = jnp.ones((P, 1), jnp.float32)

    def col_b(v):
        return _dg(v, ones_col)

    def addpq(a, b):
        return _dg(jnp.concatenate([a, ones_col], axis=1),
                   jnp.concatenate([ones_col, b], axis=1))

    sq = jnp.sum(xg * xg, axis=1, keepdims=True)
    xgT = xg.T
    sq_row = jnp.sum(xgT * xgT, axis=0, keepdims=True)
    gram2 = _dg(xg, -2.0 * xg)
    d2 = (col_b(sq) + gram2) + sq_row
    iota_f = jax.lax.broadcasted_iota(jnp.int32, (P, P), 1).astype(jnp.float32)
    inf = jnp.float32(jnp.inf)
    d2m = d2
    for _ in range(K):
        idxf = jnp.argmin(d2m, axis=1, keepdims=True).astype(jnp.float32)
        sel = iota_f == col_b(idxf)
        d2m = jnp.where(sel, inf, d2m)
    bool_mask = d2m == inf
    maskf = jnp.where(bool_mask, 1.0, 0.0)

    y = xg + _mm(maskf, xg)
    y = _leaky(_mm(y, g1W0) + g1b0)
    xg = _mm(y, g1W1) + g1b1

    xl = _mm(xg, gWl)
    xr = _mm(xg, gWr)
    att4 = 0.4 * attT
    heads = []
    for hh in range(HEADS):
        lo = hh * GNN_DIM
        xl_h = xl[:, lo:lo + GNN_DIM]
        xr_h = xr[:, lo:lo + GNN_DIM]
        a_col = attT[:, hh:hh + 1]
        sr6 = 0.6 * _mm(xr_h, a_col)
        sl6 = 0.6 * _mm(xl_h, a_col)
        e = addpq(sr6, sl6)
        for dd in range(GNN_DIM):
            c = lo + dd
            z = addpq(xr[:, c:c + 1], xl[:, c:c + 1])
            e = e + att4[dd:dd + 1, hh:hh + 1] * jnp.abs(z)
        e_masked = jnp.where(bool_mask, e, jnp.float32(-1e30))
        emax = jnp.max(e_masked, axis=1, keepdims=True)
        ee = jnp.exp(e_masked - col_b(emax))
        den = jnp.sum(ee, axis=1, keepdims=True)
        rec = 1.0 / (den + 1e-16)
        heads.append(rec * _mm(ee, xl_h))
    xg = jnp.concatenate(heads, axis=1) + gb

    y = xg + _mm(maskf, xg)
    y = _leaky(_mm(y, g2W0) + g2b0)
    xg = _mm(y, g2W1) + g2b1

    gnn = jnp.sum(xg, axis=0, keepdims=True)
    fin = jnp.concatenate([r, cond, gnn], axis=1)
    o = _leaky(_mm(fin, fW0) + fb0)
    o = _mm(o, fW1) + fb1
    return o


def _block_kernel(x_ref, cond_ref, *refs):
    out_ref = refs[-1]
    w = tuple(ref[...] for ref in refs[:-1])
    (cW0, cb0, cW1, cb1, cW2, cb2,
     rW0, rb0, rW1, rb1,
     lW0, lb0, lW1, lb1) = w[:14]

    x_all = x_ref[...]
    f = x_all
    for Wt, b in ((cW0, cb0), (cW1, cb1), (cW2, cb2)):
        f = _leaky(_mm(f, Wt) + b)
    conds, rs, cond_rows, r_rows = [], [], [], []
    for g in range(GPB):
        feat = jnp.max(f[g * P:(g + 1) * P, :], axis=0, keepdims=True)
        r = _leaky(_mm(feat, rW0) + rb0)
        r = _mm(r, rW1) + rb1
        cond = cond_ref[g].reshape(1, NC)
        conds.append(cond)
        rs.append(r)
        cond_rows.append(jnp.broadcast_to(cond, (P, NC)))
        r_rows.append(jnp.broadcast_to(r, (P, r.shape[1])))
    h = jnp.concatenate(
        [x_all,
         jnp.concatenate(cond_rows, axis=0),
         jnp.concatenate(r_rows, axis=0)], axis=1)
    xg_all = _leaky(_mm(h, lW0) + lb0)
    xg_all = _mm(xg_all, lW1) + lb1

    outs = []
    for g in range(GPB):
        xg = xg_all[g * P:(g + 1) * P, :]
        outs.append(_graph_tail(xg, conds[g], rs[g], w).reshape(1, 1, 1))
    out_ref[...] = jnp.concatenate(outs, axis=0)


def _flatten_params(params):
    flat = []

    def lin(layers):
        for W, b in layers:
            flat.append(W.T)
            flat.append(b.reshape(1, -1))

    lin(params["conv"])
    lin(params["rgan"])
    lin(params["lin"])
    lin(params["gin1"])
    gp = params["gat"]
    flat += [gp["Wl"].T, gp["Wr"].T, gp["att"].T, gp["b"].reshape(1, -1)]
    lin(params["gin2"])
    lin(params["final"])
    return flat


def kernel(x, cond, params, batch_ids):
    del batch_ids
    flat = _flatten_params(params)

    weight_specs = [
        pl.BlockSpec(w.shape, lambda b, _r=w.ndim: (0,) * _r)
        for w in flat
    ]
    out = pl.pallas_call(
        _block_kernel,
        grid=(B // GPB,),
        in_specs=[
            pl.BlockSpec((GPB * P, NF), lambda b: (b, 0)),
            pl.BlockSpec((GPB, 1, NC), lambda b: (b, 0, 0)),
            *weight_specs,
        ],
        out_specs=pl.BlockSpec((GPB, 1, 1), lambda b: (b, 0, 0)),
        out_shape=jax.ShapeDtypeStruct((B, 1, 1), jnp.float32),
        compiler_params=pltpu.CompilerParams(
            dimension_semantics=("parallel",),
        ),
    )(x, cond.reshape(B, 1, NC), *flat)
    return out.reshape(B)
